# baseline (speedup 1.0000x reference)
import functools

import jax
import jax.numpy as jnp
from jax import lax
from jax.experimental import pallas as pl
from jax.experimental.pallas import tpu as pltpu
from jax.experimental.pallas import tpu_sc as plsc

_THRESHOLD = 0.8
_NUM_CLASSES = 80
_NPAD = 5120
_NVEC = _NPAD // 16
_NC = 2
_NS = 16
_NT = _NC * _NS
_SLICE = _NPAD // _NS
_KSLOTS = -(-_NUM_CLASSES // _NT)
_NSHARED = _NPAD + 16 * _NS

_mesh = plsc.VectorSubcoreMesh(
    core_axis_name="c", subcore_axis_name="s",
    num_cores=_NC, num_subcores=_NS)


def _sc_body(x1h, y1h, x2h, y2h, cath, outh0, outh1,
             x1v, y1v, x2v, y2v, catv,
             midx0, midx1, midx2, mx1, my1, mx2, my2, mar, keepm,
             accv, shared, sem, semc):
    cid = lax.axis_index("c")
    sid = lax.axis_index("s")
    gwid = cid * _NS + sid

    cpc = pltpu.async_copy(cath, catv, semc)
    cp1 = pltpu.async_copy(x1h, x1v, sem)
    cp2 = pltpu.async_copy(y1h, y1v, sem)
    cp3 = pltpu.async_copy(x2h, x2v, sem)
    cp4 = pltpu.async_copy(y2h, y2v, sem)

    zeros16 = jnp.zeros((16,), jnp.float32)
    iota16 = lax.iota(jnp.int32, 16)

    def zero_body(u, _):
        accv[pl.ds(pl.multiple_of(u * 16, 16), 16)] = zeros16
        return 0
    lax.fori_loop(0, _SLICE // 16, zero_body, 0)
    pltpu.sync_copy(accv, shared.at[pl.ds(sid * _SLICE, _SLICE)])
    plsc.subcore_barrier()
    cpc.wait()

    cls = [gwid + _NT * k for k in range(_KSLOTS)]
    midxs = [midx0, midx1, midx2]

    def scan_body(v, cnts):
        out = cnts
        for half in range(2):
            off = pl.multiple_of(v * 32 + half * 16, 16)
            c16 = catv[pl.ds(off, 16)]
            gidx = off + iota16
            nxt = []
            for k in range(_KSLOTS):
                m = c16 == cls[k]
                plsc.store_compressed(midxs[k].at[pl.ds(out[k], 16)], gidx,
                                      mask=m)
                nxt.append(out[k] + plsc.all_reduce_population_count(m)[0])
            out = tuple(nxt)
        return out
    nums = lax.fori_loop(0, _NVEC // 2, scan_body,
                         (jnp.int32(0),) * _KSLOTS)

    cp1.wait()
    cp2.wait()
    cp3.wait()
    cp4.wait()

    def process(midx, num):
        nv = (num + 15) // 16

        def gather_body(u, _):
            off = pl.multiple_of(u * 16, 16)
            valid = (off + iota16) < num
            idx16 = jnp.where(valid, midx[pl.ds(off, 16)], 0)
            gx1 = plsc.load_gather(x1v, [idx16])
            gy1 = plsc.load_gather(y1v, [idx16])
            gx2 = plsc.load_gather(x2v, [idx16])
            gy2 = plsc.load_gather(y2v, [idx16])
            mx1[pl.ds(off, 16)] = gx1
            my1[pl.ds(off, 16)] = gy1
            mx2[pl.ds(off, 16)] = gx2
            my2[pl.ds(off, 16)] = gy2
            mar[pl.ds(off, 16)] = (gx2 - gx1) * (gy2 - gy1)
            return 0
        lax.fori_loop(0, nv, gather_body, 0)

        def rowchunk_body(t, _):
            roff = pl.multiple_of(t * 16, 16)
            rx1 = mx1[pl.ds(roff, 16)]
            ry1 = my1[pl.ds(roff, 16)]
            rx2 = mx2[pl.ds(roff, 16)]
            ry2 = my2[pl.ds(roff, 16)]
            rar = mar[pl.ds(roff, 16)]
            rlanes = roff + iota16

            def colchunk_body(u, acc):
                off = pl.multiple_of(u * 16, 16)
                cx1 = mx1[pl.ds(off, 16)]
                cy1 = my1[pl.ds(off, 16)]
                cx2 = mx2[pl.ds(off, 16)]
                cy2 = my2[pl.ds(off, 16)]
                car = mar[pl.ds(off, 16)]
                clanes = off + iota16
                car_m = jnp.where(clanes < num, car, jnp.float32(0.0))
                for r in range(16):
                    if r == 0:
                        gx1, gy1, gx2, gy2 = cx1, cy1, cx2, cy2
                        gca, cvec = car_m, clanes
                    else:
                        p = (iota16 + r) & 15
                        gx1 = cx1.at[p].get(mode="promise_in_bounds")
                        gy1 = cy1.at[p].get(mode="promise_in_bounds")
                        gx2 = cx2.at[p].get(mode="promise_in_bounds")
                        gy2 = cy2.at[p].get(mode="promise_in_bounds")
                        gca = car_m.at[p].get(mode="promise_in_bounds")
                        cvec = off + p
                    ok = ((gx1 >= rx1) & (gy1 >= ry1)
                          & (gx2 <= rx2) & (gy2 <= ry2)
                          & (rlanes != cvec))
                    acc = acc + jnp.where(ok, gca, jnp.float32(0.0))
                return acc

            acc = lax.fori_loop(0, nv, colchunk_body, zeros16)
            keep16 = jnp.where(acc <= _THRESHOLD * (rar + 1e-9),
                               jnp.float32(1.0), jnp.float32(0.0))
            keepm[pl.ds(roff, 16)] = keep16
            return 0
        lax.fori_loop(0, nv, rowchunk_body, 0)

        dummy = _NPAD + sid * 16 + iota16

        def scat_body(u, _):
            off = pl.multiple_of(u * 16, 16)
            valid = (off + iota16) < num
            idx16 = jnp.where(valid, midx[pl.ds(off, 16)], dummy)
            pltpu.sync_copy(keepm.at[pl.ds(off, 16)], shared.at[idx16],
                            add=True)
            return 0
        lax.fori_loop(0, nv, scat_body, 0)

    for k in range(_KSLOTS):
        process(midxs[k], nums[k])

    plsc.subcore_barrier()

    base = sid * _SLICE
    pltpu.sync_copy(shared.at[pl.ds(base, _SLICE)], accv)

    @pl.when(cid == 0)
    def _():
        pltpu.sync_copy(accv, outh0.at[pl.ds(base, _SLICE)])

    @pl.when(cid == 1)
    def _():
        pltpu.sync_copy(accv, outh1.at[pl.ds(base, _SLICE)])


_sc_filter = functools.partial(
    pl.kernel,
    out_type=[jax.ShapeDtypeStruct((_NPAD,), jnp.float32),
              jax.ShapeDtypeStruct((_NPAD,), jnp.float32)],
    mesh=_mesh,
    compiler_params=pltpu.CompilerParams(needs_layout_passes=False),
    scratch_types=[
        pltpu.VMEM((_NPAD,), jnp.float32),
        pltpu.VMEM((_NPAD,), jnp.float32),
        pltpu.VMEM((_NPAD,), jnp.float32),
        pltpu.VMEM((_NPAD,), jnp.float32),
        pltpu.VMEM((_NPAD,), jnp.int32),
        pltpu.VMEM((_NPAD,), jnp.int32),
        pltpu.VMEM((_NPAD,), jnp.int32),
        pltpu.VMEM((_NPAD,), jnp.int32),
        pltpu.VMEM((_NPAD,), jnp.float32),
        pltpu.VMEM((_NPAD,), jnp.float32),
        pltpu.VMEM((_NPAD,), jnp.float32),
        pltpu.VMEM((_NPAD,), jnp.float32),
        pltpu.VMEM((_NPAD,), jnp.float32),
        pltpu.VMEM((_NPAD,), jnp.float32),
        pltpu.VMEM((_SLICE,), jnp.float32),
        pltpu.VMEM_SHARED((_NSHARED,), jnp.float32),
        pltpu.SemaphoreType.DMA,
        pltpu.SemaphoreType.DMA,
    ],
)(_sc_body)


def kernel(boxes, scores, category_ids):
    n = boxes.shape[0]
    cat = category_ids.astype(jnp.int32)
    pad = _NPAD - n
    bp = jnp.pad(boxes, ((0, pad), (0, 0)))
    cp = jnp.pad(cat, (0, pad), constant_values=-1)
    x1 = bp[:, 0]
    y1 = bp[:, 1]
    x2 = bp[:, 2]
    y2 = bp[:, 3]

    p0, p1 = _sc_filter(x1, y1, x2, y2, cp)
    keep = (p0 + p1)[:n]
    box5 = jnp.concatenate([boxes, scores[:, None]], axis=1)
    return box5 * keep[:, None]


# speedup vs baseline: 1.0285x; 1.0285x over previous
import functools

import jax
import jax.numpy as jnp
from jax import lax
from jax.experimental import pallas as pl
from jax.experimental.pallas import tpu as pltpu
from jax.experimental.pallas import tpu_sc as plsc

_THRESHOLD = 0.8
_NUM_CLASSES = 80
_NPAD = 5120
_NVEC = _NPAD // 16
_NC = 2
_NS = 16
_NT = _NC * _NS
_SLICE = _NPAD // _NS
_KSLOTS = -(-_NUM_CLASSES // _NT)
_NSHARED = _NPAD + 16 * _NS

_mesh = plsc.VectorSubcoreMesh(
    core_axis_name="c", subcore_axis_name="s",
    num_cores=_NC, num_subcores=_NS)


def _sc_body(x1h, y1h, x2h, y2h, cath, outh0, outh1,
             x1v, y1v, x2v, y2v, catv,
             midx0, midx1, midx2, mx1, my1, mx2, my2, mar, keepm,
             accv, shared, sem, semc):
    cid = lax.axis_index("c")
    sid = lax.axis_index("s")
    gwid = cid * _NS + sid

    cpc = pltpu.async_copy(cath, catv, semc)
    cp1 = pltpu.async_copy(x1h, x1v, sem)
    cp2 = pltpu.async_copy(y1h, y1v, sem)
    cp3 = pltpu.async_copy(x2h, x2v, sem)
    cp4 = pltpu.async_copy(y2h, y2v, sem)

    zeros16 = jnp.zeros((16,), jnp.float32)
    iota16 = lax.iota(jnp.int32, 16)

    def zero_body(u, _):
        accv[pl.ds(pl.multiple_of(u * 16, 16), 16)] = zeros16
        return 0
    lax.fori_loop(0, _SLICE // 16, zero_body, 0)
    pltpu.sync_copy(accv, shared.at[pl.ds(sid * _SLICE, _SLICE)])
    with jax.named_scope("zeroed"):
        plsc.subcore_barrier()
    with jax.named_scope("catv_wait"):
        cpc.wait()

    cls = [gwid + _NT * k for k in range(_KSLOTS)]
    midxs = [midx0, midx1, midx2]

    def scan_body(v, cnts):
        off = pl.multiple_of(v * 16, 16)
        c16 = catv[pl.ds(off, 16)]
        gidx = off + iota16
        out = []
        for k in range(_KSLOTS):
            m = c16 == cls[k]
            plsc.store_compressed(midxs[k].at[pl.ds(cnts[k], 16)], gidx,
                                  mask=m)
            out.append(cnts[k] + plsc.all_reduce_population_count(m)[0])
        return tuple(out)
    with jax.named_scope("scan"):
        nums = lax.fori_loop(0, _NVEC, scan_body,
                             (jnp.int32(0),) * _KSLOTS)

    cp1.wait()
    cp2.wait()
    cp3.wait()
    cp4.wait()

    def process(midx, num):
        nv = (num + 15) // 16

        def gather_body(u, _):
            off = pl.multiple_of(u * 16, 16)
            valid = (off + iota16) < num
            idx16 = jnp.where(valid, midx[pl.ds(off, 16)], 0)
            gx1 = plsc.load_gather(x1v, [idx16])
            gy1 = plsc.load_gather(y1v, [idx16])
            gx2 = plsc.load_gather(x2v, [idx16])
            gy2 = plsc.load_gather(y2v, [idx16])
            mx1[pl.ds(off, 16)] = gx1
            my1[pl.ds(off, 16)] = gy1
            mx2[pl.ds(off, 16)] = gx2
            my2[pl.ds(off, 16)] = gy2
            mar[pl.ds(off, 16)] = (gx2 - gx1) * (gy2 - gy1)
            return 0
        lax.fori_loop(0, nv, gather_body, 0)

        def rowchunk_body(t, _):
            roff = pl.multiple_of(t * 16, 16)
            rx1 = mx1[pl.ds(roff, 16)]
            ry1 = my1[pl.ds(roff, 16)]
            rx2 = mx2[pl.ds(roff, 16)]
            ry2 = my2[pl.ds(roff, 16)]
            rar = mar[pl.ds(roff, 16)]
            rlanes = roff + iota16

            def colchunk_body(u, acc):
                off = pl.multiple_of(u * 16, 16)
                cx1 = mx1[pl.ds(off, 16)]
                cy1 = my1[pl.ds(off, 16)]
                cx2 = mx2[pl.ds(off, 16)]
                cy2 = my2[pl.ds(off, 16)]
                car = mar[pl.ds(off, 16)]
                clanes = off + iota16
                car_m = jnp.where(clanes < num, car, jnp.float32(0.0))
                for r in range(16):
                    if r == 0:
                        gx1, gy1, gx2, gy2 = cx1, cy1, cx2, cy2
                        gca, cvec = car_m, clanes
                    else:
                        p = (iota16 + r) & 15
                        gx1 = cx1.at[p].get(mode="promise_in_bounds")
                        gy1 = cy1.at[p].get(mode="promise_in_bounds")
                        gx2 = cx2.at[p].get(mode="promise_in_bounds")
                        gy2 = cy2.at[p].get(mode="promise_in_bounds")
                        gca = car_m.at[p].get(mode="promise_in_bounds")
                        cvec = off + p
                    ok = ((gx1 >= rx1) & (gy1 >= ry1)
                          & (gx2 <= rx2) & (gy2 <= ry2)
                          & (rlanes != cvec))
                    acc = acc + jnp.where(ok, gca, jnp.float32(0.0))
                return acc

            acc = lax.fori_loop(0, nv, colchunk_body, zeros16)
            keep16 = jnp.where(acc <= _THRESHOLD * (rar + 1e-9),
                               jnp.float32(1.0), jnp.float32(0.0))
            keepm[pl.ds(roff, 16)] = keep16
            return 0
        lax.fori_loop(0, nv, rowchunk_body, 0)

        dummy = _NPAD + sid * 16 + iota16

        def scat_body(u, _):
            off = pl.multiple_of(u * 16, 16)
            valid = (off + iota16) < num
            idx16 = jnp.where(valid, midx[pl.ds(off, 16)], dummy)
            pltpu.sync_copy(keepm.at[pl.ds(off, 16)], shared.at[idx16],
                            add=True)
            return 0
        lax.fori_loop(0, nv, scat_body, 0)

    with jax.named_scope("coord_wait"):
        pass
    for k in range(_KSLOTS):
        with jax.named_scope(f"class{k}"):
            process(midxs[k], nums[k])

    with jax.named_scope("bar2"):
        plsc.subcore_barrier()

    base = sid * _SLICE
    with jax.named_scope("readback"):
        pltpu.sync_copy(shared.at[pl.ds(base, _SLICE)], accv)
    with jax.named_scope("out"):
        pass

    @pl.when(cid == 0)
    def _():
        pltpu.sync_copy(accv, outh0.at[pl.ds(base, _SLICE)])

    @pl.when(cid == 1)
    def _():
        pltpu.sync_copy(accv, outh1.at[pl.ds(base, _SLICE)])


_sc_filter = functools.partial(
    pl.kernel,
    out_type=[jax.ShapeDtypeStruct((_NPAD,), jnp.float32),
              jax.ShapeDtypeStruct((_NPAD,), jnp.float32)],
    mesh=_mesh,
    compiler_params=pltpu.CompilerParams(needs_layout_passes=False),
    scratch_types=[
        pltpu.VMEM((_NPAD,), jnp.float32),
        pltpu.VMEM((_NPAD,), jnp.float32),
        pltpu.VMEM((_NPAD,), jnp.float32),
        pltpu.VMEM((_NPAD,), jnp.float32),
        pltpu.VMEM((_NPAD,), jnp.int32),
        pltpu.VMEM((_NPAD,), jnp.int32),
        pltpu.VMEM((_NPAD,), jnp.int32),
        pltpu.VMEM((_NPAD,), jnp.int32),
        pltpu.VMEM((_NPAD,), jnp.float32),
        pltpu.VMEM((_NPAD,), jnp.float32),
        pltpu.VMEM((_NPAD,), jnp.float32),
        pltpu.VMEM((_NPAD,), jnp.float32),
        pltpu.VMEM((_NPAD,), jnp.float32),
        pltpu.VMEM((_NPAD,), jnp.float32),
        pltpu.VMEM((_SLICE,), jnp.float32),
        pltpu.VMEM_SHARED((_NSHARED,), jnp.float32),
        pltpu.SemaphoreType.DMA,
        pltpu.SemaphoreType.DMA,
    ],
)(_sc_body)


def kernel(boxes, scores, category_ids):
    n = boxes.shape[0]
    cat = category_ids.astype(jnp.int32)
    pad = _NPAD - n
    bp = jnp.pad(boxes, ((0, pad), (0, 0)))
    cp = jnp.pad(cat, (0, pad), constant_values=-1)
    x1 = bp[:, 0]
    y1 = bp[:, 1]
    x2 = bp[:, 2]
    y2 = bp[:, 3]

    p0, p1 = _sc_filter(x1, y1, x2, y2, cp)
    keep = (p0 + p1)[:n]
    box5 = jnp.concatenate([boxes, scores[:, None]], axis=1)
    return box5 * keep[:, None]


# speedup vs baseline: 1.0289x; 1.0004x over previous
import functools

import jax
import jax.numpy as jnp
from jax import lax
from jax.experimental import pallas as pl
from jax.experimental.pallas import tpu as pltpu
from jax.experimental.pallas import tpu_sc as plsc

_THRESHOLD = 0.8
_NUM_CLASSES = 80
_NPAD = 5120
_NVEC = _NPAD // 16
_NC = 2
_NS = 16
_NT = _NC * _NS
_SLICE = _NPAD // _NS
_KSLOTS = -(-_NUM_CLASSES // _NT)
_NSHARED = _NPAD + 16 * _NS

_mesh = plsc.VectorSubcoreMesh(
    core_axis_name="c", subcore_axis_name="s",
    num_cores=_NC, num_subcores=_NS)


def _sc_body(x1h, y1h, x2h, y2h, cath, outh0, outh1,
             x1v, y1v, x2v, y2v, catv,
             midx0, midx1, midx2, mx1, my1, mx2, my2, mar, keepm,
             accv, shared, sem, semc):
    cid = lax.axis_index("c")
    sid = lax.axis_index("s")
    gwid = cid * _NS + sid

    cpc = pltpu.async_copy(cath, catv, semc)
    cp1 = pltpu.async_copy(x1h, x1v, sem)
    cp2 = pltpu.async_copy(y1h, y1v, sem)
    cp3 = pltpu.async_copy(x2h, x2v, sem)
    cp4 = pltpu.async_copy(y2h, y2v, sem)

    zeros16 = jnp.zeros((16,), jnp.float32)
    iota16 = lax.iota(jnp.int32, 16)

    def zero_body(u, _):
        accv[pl.ds(pl.multiple_of(u * 16, 16), 16)] = zeros16
        return 0
    lax.fori_loop(0, _SLICE // 16, zero_body, 0)
    pltpu.sync_copy(accv, shared.at[pl.ds(sid * _SLICE, _SLICE)])
    plsc.subcore_barrier()
    cpc.wait()

    cls = [gwid + _NT * k for k in range(_KSLOTS)]
    midxs = [midx0, midx1, midx2]

    def scan_body(v, cnts):
        off = pl.multiple_of(v * 16, 16)
        c16 = catv[pl.ds(off, 16)]
        gidx = off + iota16
        out = []
        for k in range(_KSLOTS):
            m = c16 == cls[k]
            plsc.store_compressed(midxs[k].at[pl.ds(cnts[k], 16)], gidx,
                                  mask=m)
            out.append(cnts[k] + plsc.all_reduce_population_count(m)[0])
        return tuple(out)
    nums = lax.fori_loop(0, _NVEC, scan_body,
                         (jnp.int32(0),) * _KSLOTS)

    cp1.wait()
    cp2.wait()
    cp3.wait()
    cp4.wait()

    def process(midx, num):
        nv = (num + 15) // 16

        def gather_body(u, _):
            off = pl.multiple_of(u * 16, 16)
            valid = (off + iota16) < num
            idx16 = jnp.where(valid, midx[pl.ds(off, 16)], 0)
            gx1 = plsc.load_gather(x1v, [idx16])
            gy1 = plsc.load_gather(y1v, [idx16])
            gx2 = plsc.load_gather(x2v, [idx16])
            gy2 = plsc.load_gather(y2v, [idx16])
            mx1[pl.ds(off, 16)] = gx1
            my1[pl.ds(off, 16)] = gy1
            mx2[pl.ds(off, 16)] = gx2
            my2[pl.ds(off, 16)] = gy2
            mar[pl.ds(off, 16)] = (gx2 - gx1) * (gy2 - gy1)
            return 0
        lax.fori_loop(0, nv, gather_body, 0)

        def rowchunk_body(t, _):
            roff = pl.multiple_of(t * 16, 16)
            rx1 = mx1[pl.ds(roff, 16)]
            ry1 = my1[pl.ds(roff, 16)]
            rx2 = mx2[pl.ds(roff, 16)]
            ry2 = my2[pl.ds(roff, 16)]
            rar = mar[pl.ds(roff, 16)]
            rlanes = roff + iota16

            def colchunk_body(u, acc):
                off = pl.multiple_of(u * 16, 16)
                cx1 = mx1[pl.ds(off, 16)]
                cy1 = my1[pl.ds(off, 16)]
                cx2 = mx2[pl.ds(off, 16)]
                cy2 = my2[pl.ds(off, 16)]
                car = mar[pl.ds(off, 16)]
                clanes = off + iota16
                car_m = jnp.where(clanes < num, car, jnp.float32(0.0))
                for r in range(16):
                    if r == 0:
                        gx1, gy1, gx2, gy2 = cx1, cy1, cx2, cy2
                        gca, cvec = car_m, clanes
                    else:
                        p = (iota16 + r) & 15
                        gx1 = cx1.at[p].get(mode="promise_in_bounds")
                        gy1 = cy1.at[p].get(mode="promise_in_bounds")
                        gx2 = cx2.at[p].get(mode="promise_in_bounds")
                        gy2 = cy2.at[p].get(mode="promise_in_bounds")
                        gca = car_m.at[p].get(mode="promise_in_bounds")
                        cvec = off + p
                    ok = ((gx1 >= rx1) & (gy1 >= ry1)
                          & (gx2 <= rx2) & (gy2 <= ry2)
                          & (rlanes != cvec))
                    acc = acc + jnp.where(ok, gca, jnp.float32(0.0))
                return acc

            acc = lax.fori_loop(0, nv, colchunk_body, zeros16)
            keep16 = jnp.where(acc <= _THRESHOLD * (rar + 1e-9),
                               jnp.float32(1.0), jnp.float32(0.0))
            keepm[pl.ds(roff, 16)] = keep16
            return 0
        lax.fori_loop(0, nv, rowchunk_body, 0)

        dummy = _NPAD + sid * 16 + iota16

        def scat_body(u, _):
            off = pl.multiple_of(u * 16, 16)
            valid = (off + iota16) < num
            idx16 = jnp.where(valid, midx[pl.ds(off, 16)], dummy)
            pltpu.sync_copy(keepm.at[pl.ds(off, 16)], shared.at[idx16],
                            add=True)
            return 0
        lax.fori_loop(0, nv, scat_body, 0)

    for k in range(_KSLOTS):
        process(midxs[k], nums[k])

    plsc.subcore_barrier()

    base = sid * _SLICE
    pltpu.sync_copy(shared.at[pl.ds(base, _SLICE)], accv)

    @pl.when(cid == 0)
    def _():
        pltpu.sync_copy(accv, outh0.at[pl.ds(base, _SLICE)])

    @pl.when(cid == 1)
    def _():
        pltpu.sync_copy(accv, outh1.at[pl.ds(base, _SLICE)])


_sc_filter = functools.partial(
    pl.kernel,
    out_type=[jax.ShapeDtypeStruct((_NPAD,), jnp.float32),
              jax.ShapeDtypeStruct((_NPAD,), jnp.float32)],
    mesh=_mesh,
    compiler_params=pltpu.CompilerParams(needs_layout_passes=False),
    scratch_types=[
        pltpu.VMEM((_NPAD,), jnp.float32),
        pltpu.VMEM((_NPAD,), jnp.float32),
        pltpu.VMEM((_NPAD,), jnp.float32),
        pltpu.VMEM((_NPAD,), jnp.float32),
        pltpu.VMEM((_NPAD,), jnp.int32),
        pltpu.VMEM((_NPAD,), jnp.int32),
        pltpu.VMEM((_NPAD,), jnp.int32),
        pltpu.VMEM((_NPAD,), jnp.int32),
        pltpu.VMEM((_NPAD,), jnp.float32),
        pltpu.VMEM((_NPAD,), jnp.float32),
        pltpu.VMEM((_NPAD,), jnp.float32),
        pltpu.VMEM((_NPAD,), jnp.float32),
        pltpu.VMEM((_NPAD,), jnp.float32),
        pltpu.VMEM((_NPAD,), jnp.float32),
        pltpu.VMEM((_SLICE,), jnp.float32),
        pltpu.VMEM_SHARED((_NSHARED,), jnp.float32),
        pltpu.SemaphoreType.DMA,
        pltpu.SemaphoreType.DMA,
    ],
)(_sc_body)


def kernel(boxes, scores, category_ids):
    n = boxes.shape[0]
    cat = category_ids.astype(jnp.int32)
    pad = _NPAD - n
    bp = jnp.pad(boxes, ((0, pad), (0, 0)))
    cp = jnp.pad(cat, (0, pad), constant_values=-1)
    x1 = bp[:, 0]
    y1 = bp[:, 1]
    x2 = bp[:, 2]
    y2 = bp[:, 3]

    p0, p1 = _sc_filter(x1, y1, x2, y2, cp)
    keep = (p0 + p1)[:n]
    box5 = jnp.concatenate([boxes, scores[:, None]], axis=1)
    return box5 * keep[:, None]
